# baseline (speedup 1.0000x reference)
import jax
import jax.numpy as jnp
from jax import lax
from jax.experimental import pallas as pl
from jax.experimental.pallas import tpu as pltpu
from jax.experimental.pallas import tpu_sc as plsc

B = 1024
F = 81920
FSC = 16384
FB = 2048
CH = 512
NCH = FSC // CH
NTILES = 32
RPT = B // NTILES
RG = 8


def _sc_compute(xbuf, w0buf, acc_ref):
    for g in range(RPT // RG):
        rows = [g * RG + r for r in range(RG)]

        def step(t, carry):
            xs = [xbuf[rw, pl.ds(t * 16, 16)] for rw in rows]
            ws = [w0buf[j, pl.ds(t * 16, 16)] for j in range(4)]
            out = []
            idx = 0
            for r in range(RG):
                for j in range(4):
                    out.append(carry[idx] + xs[r] * ws[j])
                    idx += 1
            return tuple(out)

        init = tuple(acc_ref[rw, j] for rw in rows for j in range(4))
        fin = lax.fori_loop(0, CH // 16, step, init)
        idx = 0
        for rw in rows:
            for j in range(4):
                acc_ref[rw, j] = fin[idx]
                idx += 1


def _sc_body(white_hbm, black_hbm, w0_hbm, partw_hbm, partb_hbm,
             wbuf, bbuf, w0buf, accw, accb, wsem, bsem, w0sem):
    c = lax.axis_index("c")
    s = lax.axis_index("s")
    wid = s * 2 + c
    base = wid * RPT
    fmax = (NCH - 1) * CH

    def issue(k, slot):
        f0 = jnp.minimum(k * CH, fmax)
        pltpu.async_copy(w0_hbm.at[:, pl.ds(f0, CH)], w0buf.at[slot],
                         w0sem.at[slot])
        pltpu.async_copy(white_hbm.at[pl.ds(base, RPT), pl.ds(f0, CH)],
                         wbuf.at[slot], wsem.at[slot])
        pltpu.async_copy(black_hbm.at[pl.ds(base, RPT), pl.ds(f0, CH)],
                         bbuf.at[slot], bsem.at[slot])

    def drain(slot):
        pltpu.make_async_copy(w0_hbm.at[:, pl.ds(0, CH)], w0buf.at[slot],
                              w0sem.at[slot]).wait()
        pltpu.make_async_copy(white_hbm.at[pl.ds(base, RPT), pl.ds(0, CH)],
                              wbuf.at[slot], wsem.at[slot]).wait()
        pltpu.make_async_copy(black_hbm.at[pl.ds(base, RPT), pl.ds(0, CH)],
                              bbuf.at[slot], bsem.at[slot]).wait()

    zero = jnp.zeros((16,), jnp.float32)
    for r in range(RPT):
        for j in range(4):
            accw[r, j] = zero
            accb[r, j] = zero

    issue(0, 0)

    def pair(kk, _):
        k0 = 2 * kk
        issue(k0 + 1, 1)
        drain(0)
        _sc_compute(wbuf.at[0], w0buf.at[0], accw)
        _sc_compute(bbuf.at[0], w0buf.at[0], accb)
        issue(k0 + 2, 0)
        drain(1)
        _sc_compute(wbuf.at[1], w0buf.at[1], accw)
        _sc_compute(bbuf.at[1], w0buf.at[1], accb)
        return 0

    lax.fori_loop(0, NCH // 2, pair, 0)
    drain(0)

    pltpu.sync_copy(accw, partw_hbm.at[pl.ds(base, RPT)])
    pltpu.sync_copy(accb, partb_hbm.at[pl.ds(base, RPT)])


def _sc_partials(white_features, black_features, W0):
    mesh = plsc.VectorSubcoreMesh(core_axis_name="c", subcore_axis_name="s")
    part = jax.ShapeDtypeStruct((B, 4, 16), jnp.float32)
    return pl.kernel(
        _sc_body,
        out_type=(part, part),
        mesh=mesh,
        scratch_types=[
            pltpu.VMEM((2, RPT, CH), jnp.float32),
            pltpu.VMEM((2, RPT, CH), jnp.float32),
            pltpu.VMEM((2, 4, CH), jnp.float32),
            pltpu.VMEM((RPT, 4, 16), jnp.float32),
            pltpu.VMEM((RPT, 4, 16), jnp.float32),
            pltpu.SemaphoreType.DMA((2,)),
            pltpu.SemaphoreType.DMA((2,)),
            pltpu.SemaphoreType.DMA((2,)),
        ],
    )(white_features, black_features, W0)


def _tc_kernel(white_ref, black_ref, w0_ref, accw_out, accb_out,
               accw_ref, accb_ref):
    i = pl.program_id(0)
    nsteps = pl.num_programs(0)

    @pl.when(i == 0)
    def _init():
        accw_ref[...] = jnp.zeros_like(accw_ref)
        accb_ref[...] = jnp.zeros_like(accb_ref)

    dn = (((1,), (1,)), ((), ()))
    w0b = w0_ref[...].astype(jnp.bfloat16)
    accw_ref[...] += lax.dot_general(
        white_ref[...].astype(jnp.bfloat16), w0b, dn,
        preferred_element_type=jnp.float32)
    accb_ref[...] += lax.dot_general(
        black_ref[...].astype(jnp.bfloat16), w0b, dn,
        preferred_element_type=jnp.float32)

    @pl.when(i == nsteps - 1)
    def _out():
        accw_out[...] = accw_ref[...]
        accb_out[...] = accb_ref[...]


def _tc_partials(white_features, black_features, W0):
    off = FSC // FB
    nsteps = (F - FSC) // FB
    out = jax.ShapeDtypeStruct((B, 4), jnp.float32)
    return pl.pallas_call(
        _tc_kernel,
        grid=(nsteps,),
        in_specs=[
            pl.BlockSpec((B, FB), lambda i: (0, i + off)),
            pl.BlockSpec((B, FB), lambda i: (0, i + off)),
            pl.BlockSpec((4, FB), lambda i: (0, i + off)),
        ],
        out_specs=[pl.BlockSpec((B, 4), lambda i: (0, 0)),
                   pl.BlockSpec((B, 4), lambda i: (0, 0))],
        out_shape=(out, out),
        scratch_shapes=[pltpu.VMEM((B, 4), jnp.float32),
                        pltpu.VMEM((B, 4), jnp.float32)],
    )(white_features, black_features, W0)


def _epi_kernel(tcw_ref, tcb_ref, scw_ref, scb_ref, turn_ref, score_ref,
                b0_ref, w1_ref, b1_ref, w2_ref, b2_ref, loss_ref):
    def lane_sum(ref):
        x = ref[...]
        cols = [jnp.sum(x[:, j * 16:(j + 1) * 16], axis=1, keepdims=True)
                for j in range(4)]
        return jnp.concatenate(cols, axis=1)

    b0 = b0_ref[...]
    w = tcw_ref[...] + lane_sum(scw_ref) + b0
    b = tcb_ref[...] + lane_sum(scb_ref) + b0
    turn = turn_ref[...]
    wb = jnp.concatenate([w, b], axis=1)
    bw = jnp.concatenate([b, w], axis=1)
    accum = turn * wb + (1.0 - turn) * bw
    l1_x = jnp.clip(accum, 0.0, 1.0)
    dn = (((1,), (1,)), ((), ()))
    l2 = lax.dot_general(l1_x, w1_ref[...], dn,
                         preferred_element_type=jnp.float32) + b1_ref[...]
    l2_x = jnp.clip(l2, 0.0, 1.0)
    model = jnp.sum(l2_x * w2_ref[...], axis=1, keepdims=True) + b2_ref[...]
    wdl_model = jax.nn.sigmoid(model / 400.0)
    wdl_target = jax.nn.sigmoid(score_ref[...] / 400.0)
    loss_ref[...] = (wdl_model - wdl_target) ** 2


def _epilogue(tcw, tcb, scw, scb, turn, score, b0, W1, b1, W2, b2):
    return pl.pallas_call(
        _epi_kernel,
        out_shape=jax.ShapeDtypeStruct((B, 1), jnp.float32),
    )(tcw, tcb, scw.reshape(B, 64), scb.reshape(B, 64), turn, score,
      b0, W1, b1, W2, b2)


@jax.jit
def _nnue(white_features, black_features, turn, score,
          W0, b0, W1, b1, W2, b2):
    scw, scb = _sc_partials(white_features, black_features, W0)
    tcw, tcb = _tc_partials(white_features, black_features, W0)
    return _epilogue(tcw, tcb, scw, scb, turn, score, b0, W1, b1, W2, b2)


def kernel(white_features, black_features, turn, score, result,
           W0, b0, W1, b1, W2, b2):
    del result
    return _nnue(white_features, black_features, turn, score,
                 W0, b0.reshape(1, 4), W1, b1.reshape(1, 8),
                 W2.reshape(1, 8), b2.reshape(1, 1))


# speedup vs baseline: 1.2233x; 1.2233x over previous
import jax
import jax.numpy as jnp
from jax.experimental import pallas as pl
from jax.experimental.pallas import tpu as pltpu

B = 1024
F = 81920
FB = 2048


def _nnue_kernel(white_ref, black_ref, turn_ref, score_ref,
                 w0_ref, b0_ref, w1_ref, b1_ref, w2_ref, b2_ref,
                 loss_ref, accw_ref, accb_ref):
    i = pl.program_id(0)
    nsteps = pl.num_programs(0)

    @pl.when(i == 0)
    def _init():
        accw_ref[...] = jnp.zeros_like(accw_ref)
        accb_ref[...] = jnp.zeros_like(accb_ref)

    dn = (((1,), (1,)), ((), ()))
    w0 = w0_ref[...]
    wpart = jax.lax.dot_general(white_ref[...], w0, dn,
                                preferred_element_type=jnp.float32)
    bpart = jax.lax.dot_general(black_ref[...], w0, dn,
                                preferred_element_type=jnp.float32)
    accw_ref[...] += wpart
    accb_ref[...] += bpart

    @pl.when(i == nsteps - 1)
    def _epilogue():
        b0 = b0_ref[...]
        w = accw_ref[...] + b0
        b = accb_ref[...] + b0
        turn = turn_ref[...]
        wb = jnp.concatenate([w, b], axis=1)
        bw = jnp.concatenate([b, w], axis=1)
        accum = turn * wb + (1.0 - turn) * bw
        l1_x = jnp.clip(accum, 0.0, 1.0)
        dn2 = (((1,), (1,)), ((), ()))
        l2 = jax.lax.dot_general(l1_x, w1_ref[...], dn2,
                                 preferred_element_type=jnp.float32) + b1_ref[...]
        l2_x = jnp.clip(l2, 0.0, 1.0)
        model = jnp.sum(l2_x * w2_ref[...], axis=1,
                        keepdims=True) + b2_ref[...]
        wdl_model = jax.nn.sigmoid(model / 400.0)
        wdl_target = jax.nn.sigmoid(score_ref[...] / 400.0)
        loss_ref[...] = (wdl_model - wdl_target) ** 2


@jax.jit
def _nnue(white_features, black_features, turn, score,
          W0, b0, W1, b1, W2, b2):
    grid = (F // FB,)
    return pl.pallas_call(
        _nnue_kernel,
        grid=grid,
        in_specs=[
            pl.BlockSpec((B, FB), lambda i: (0, i)),
            pl.BlockSpec((B, FB), lambda i: (0, i)),
            pl.BlockSpec((B, 1), lambda i: (0, 0)),
            pl.BlockSpec((B, 1), lambda i: (0, 0)),
            pl.BlockSpec((4, FB), lambda i: (0, i)),
            pl.BlockSpec((1, 4), lambda i: (0, 0)),
            pl.BlockSpec((8, 8), lambda i: (0, 0)),
            pl.BlockSpec((1, 8), lambda i: (0, 0)),
            pl.BlockSpec((1, 8), lambda i: (0, 0)),
            pl.BlockSpec((1, 1), lambda i: (0, 0)),
        ],
        out_specs=pl.BlockSpec((B, 1), lambda i: (0, 0)),
        out_shape=jax.ShapeDtypeStruct((B, 1), jnp.float32),
        scratch_shapes=[pltpu.VMEM((B, 4), jnp.float32),
                        pltpu.VMEM((B, 4), jnp.float32)],
    )(white_features, black_features, turn, score,
      W0, b0, W1, b1, W2, b2)


def kernel(white_features, black_features, turn, score, result,
           W0, b0, W1, b1, W2, b2):
    del result
    return _nnue(white_features, black_features, turn, score,
                 W0, b0.reshape(1, 4), W1, b1.reshape(1, 8),
                 W2.reshape(1, 8), b2.reshape(1, 1))
